# Initial kernel scaffold; baseline (speedup 1.0000x reference)
import jax
import jax.numpy as jnp
from jax.experimental import pallas as pl


def kernel(node_feature, params, edge_index):
    raise NotImplementedError("write your pallas kernel here")


# speedup vs baseline: 3.1155x; 3.1155x over previous
import functools

import jax
import jax.numpy as jnp
from jax import lax
from jax.experimental import pallas as pl
from jax.experimental.pallas import tpu as pltpu
from jax.experimental.pallas import tpu_sc as plsc

N = 10000
E = 320000
F = 128
NC, NS = 2, 16
NW = NC * NS
C = 128
NCHUNK = E // C
ROWS_PER_TILE = 624
TAIL_ROWS = N - NS * ROWS_PER_TILE
_PIECES = ((0, 128), (128, 128), (256, 128), (384, 128), (512, 112))

BR = 2000
BE = 1280


def _mm(h, w):
    return lax.dot_general(h, w, (((1,), (0,)), ((), ())),
                           preferred_element_type=jnp.float32)


def _full(shape):
    return pl.BlockSpec(shape, lambda i: (0,) * len(shape))


def _mlp4_body(x_ref, w1, b1, w2, b2, w3, b3, w4, b4, o_ref):
    h = x_ref[...]
    h = jnp.maximum(_mm(h, w1[...]) + b1[...], 0.0)
    h = jnp.maximum(_mm(h, w2[...]) + b2[...], 0.0)
    h = jnp.maximum(_mm(h, w3[...]) + b3[...], 0.0)
    o_ref[...] = jnp.maximum(_mm(h, w4[...]) + b4[...], 0.0)


def _tc_mlp4(x, ps):
    args = [x]
    in_specs = [pl.BlockSpec((BR, F), lambda i: (i, 0))]
    for w, b in ps:
        args += [w, b.reshape(1, F)]
        in_specs += [_full((F, F)), _full((1, F))]
    return pl.pallas_call(
        _mlp4_body,
        grid=(N // BR,),
        in_specs=in_specs,
        out_specs=pl.BlockSpec((BR, F), lambda i: (i, 0)),
        out_shape=jax.ShapeDtypeStruct((N, F), jnp.float32),
    )(*args)


def _pq_body(x_ref, wa, wb, b1, p_ref, q_ref):
    h = x_ref[...]
    p_ref[...] = _mm(h, wa[...]) + b1[...]
    q_ref[...] = _mm(h, wb[...])


def _tc_pq(x, wa, wb, b1):
    return pl.pallas_call(
        _pq_body,
        grid=(N // BR,),
        in_specs=[pl.BlockSpec((BR, F), lambda i: (i, 0)),
                  _full((F, F)), _full((F, F)), _full((1, F))],
        out_specs=(pl.BlockSpec((BR, F), lambda i: (i, 0)),
                   pl.BlockSpec((BR, F), lambda i: (i, 0))),
        out_shape=(jax.ShapeDtypeStruct((N, F), jnp.float32),
                   jax.ShapeDtypeStruct((N, F), jnp.float32)),
    )(x, wa, wb, b1.reshape(1, F))


def _edge_body(g_ref, w2, b2, w3, b3, w4, b4, o_ref):
    h = jnp.maximum(g_ref[...], 0.0)
    h = jnp.maximum(_mm(h, w2[...]) + b2[...], 0.0)
    h = jnp.maximum(_mm(h, w3[...]) + b3[...], 0.0)
    o_ref[...] = jnp.maximum(_mm(h, w4[...]) + b4[...], 0.0)


def _tc_edge(g, w2, b2, w3, b3, w4, b4):
    return pl.pallas_call(
        _edge_body,
        grid=(E // BE,),
        in_specs=[pl.BlockSpec((BE, F), lambda i: (i, 0)),
                  _full((F, F)), _full((1, F)),
                  _full((F, F)), _full((1, F)),
                  _full((F, F)), _full((1, F))],
        out_specs=pl.BlockSpec((BE, F), lambda i: (i, 0)),
        out_shape=jax.ShapeDtypeStruct((E, F), jnp.float32),
    )(g, w2, b2.reshape(1, F), w3, b3.reshape(1, F), w4, b4.reshape(1, F))


def _upd_body(x_ref, p0_ref, p1_ref, va, vb, b1, w2, b2, w3, b3, w4, b4, o_ref):
    x = x_ref[...]
    a = p0_ref[...] + p1_ref[...]
    h = jnp.maximum(_mm(x, va[...]) + _mm(a, vb[...]) + b1[...], 0.0)
    h = jnp.maximum(_mm(h, w2[...]) + b2[...], 0.0)
    h = jnp.maximum(_mm(h, w3[...]) + b3[...], 0.0)
    o_ref[...] = x + jnp.maximum(_mm(h, w4[...]) + b4[...], 0.0)


def _tc_update(x, p0, p1, va, vb, b1, w2, b2, w3, b3, w4, b4):
    row = pl.BlockSpec((BR, F), lambda i: (i, 0))
    return pl.pallas_call(
        _upd_body,
        grid=(N // BR,),
        in_specs=[row, row, row,
                  _full((F, F)), _full((F, F)), _full((1, F)),
                  _full((F, F)), _full((1, F)),
                  _full((F, F)), _full((1, F)),
                  _full((F, F)), _full((1, F))],
        out_specs=row,
        out_shape=jax.ShapeDtypeStruct((N, F), jnp.float32),
    )(x, p0, p1, va, vb, b1.reshape(1, F), w2, b2.reshape(1, F),
      w3, b3.reshape(1, F), w4, b4.reshape(1, F))


def _sc_mesh():
    return plsc.VectorSubcoreMesh(core_axis_name="c", subcore_axis_name="s",
                                  num_cores=NC, num_subcores=NS)


def _sc_gather(p, q, dst, src):
    @functools.partial(
        pl.kernel,
        out_type=jax.ShapeDtypeStruct((E, F), jnp.float32),
        mesh=_sc_mesh(),
        scratch_types=[
            pltpu.VMEM((C,), jnp.int32),
            pltpu.VMEM((C,), jnp.int32),
            pltpu.VMEM((C, F), jnp.float32),
            pltpu.VMEM((C, F), jnp.float32),
            pltpu.SemaphoreType.DMA,
            pltpu.SemaphoreType.DMA,
        ],
    )
    def k(p_hbm, q_hbm, dst_hbm, src_hbm, g_hbm, idx_d, idx_s, bufp, bufq,
          semp, semq):
        wid = lax.axis_index("s") * NC + lax.axis_index("c")
        nmine = (NCHUNK - wid + NW - 1) // NW

        def chunk(i, carry):
            base = (wid + i * NW) * C
            pltpu.sync_copy(dst_hbm.at[pl.ds(base, C)], idx_d)
            pltpu.sync_copy(src_hbm.at[pl.ds(base, C)], idx_s)
            cp1 = pltpu.async_copy(p_hbm.at[idx_d], bufp, semp)
            cp2 = pltpu.async_copy(q_hbm.at[idx_s], bufq, semq)
            cp1.wait()
            cp2.wait()

            def row(r, c2):
                for kk in range(F // 16):
                    sl = pl.ds(kk * 16, 16)
                    bufp[r, sl] = bufp[r, sl] + bufq[r, sl]
                return c2

            lax.fori_loop(0, C, row, 0)
            pltpu.sync_copy(bufp, g_hbm.at[pl.ds(base, C)])
            return carry

        lax.fori_loop(0, nmine, chunk, 0)

    return k(p, q, dst, src)


def _sc_scatter(msg, dst):
    @functools.partial(
        pl.kernel,
        out_type=jax.ShapeDtypeStruct((NC, N, F), jnp.float32),
        mesh=_sc_mesh(),
        scratch_types=[
            pltpu.VMEM((C,), jnp.int32),
            pltpu.VMEM((C, F), jnp.float32),
            pltpu.VMEM_SHARED((N, F), jnp.float32),
            pltpu.SemaphoreType.DMA,
        ],
    )
    def k(msg_hbm, dst_hbm, out_hbm, idx_v, buf, acc, sem):
        cid = lax.axis_index("c")
        sid = lax.axis_index("s")
        wid = sid * NC + cid
        nmine = (NCHUNK - wid + NW - 1) // NW

        def zrow(r, c2):
            for kk in range(F // 16):
                buf[r, pl.ds(kk * 16, 16)] = jnp.zeros((16,), jnp.float32)
            return c2

        lax.fori_loop(0, C, zrow, 0)
        tile0 = sid * ROWS_PER_TILE
        for off, sz in _PIECES:
            r0 = pl.multiple_of(tile0 + off, 8)
            pltpu.sync_copy(buf.at[pl.ds(0, sz)], acc.at[pl.ds(r0, sz)])

        @pl.when(sid == 0)
        def _zero_tail():
            pltpu.sync_copy(buf.at[pl.ds(0, TAIL_ROWS)],
                            acc.at[pl.ds(NS * ROWS_PER_TILE, TAIL_ROWS)])

        plsc.subcore_barrier()

        def chunk(i, carry):
            base = (wid + i * NW) * C
            pltpu.sync_copy(dst_hbm.at[pl.ds(base, C)], idx_v)
            pltpu.sync_copy(msg_hbm.at[pl.ds(base, C)], buf)
            pltpu.sync_copy(buf, acc.at[idx_v], add=True)
            return carry

        lax.fori_loop(0, nmine, chunk, 0)
        plsc.subcore_barrier()

        for off, sz in _PIECES:
            r0 = pl.multiple_of(tile0 + off, 8)
            pltpu.sync_copy(acc.at[pl.ds(r0, sz)], out_hbm.at[cid, pl.ds(r0, sz)])

        @pl.when(sid == 0)
        def _write_tail():
            rt = NS * ROWS_PER_TILE
            pltpu.sync_copy(acc.at[pl.ds(rt, TAIL_ROWS)],
                            out_hbm.at[cid, pl.ds(rt, TAIL_ROWS)])

    return k(msg, dst)


def kernel(node_feature, params, edge_index):
    src = edge_index[0]
    dst = edge_index[1]
    x = _tc_mlp4(node_feature, params["node_in"])
    for lp in params["layers"]:
        (w1, b1), (w2, b2), (w3, b3), (w4, b4) = lp
        w1a, w1b = w1[:F], w1[F:]
        p, q = _tc_pq(x, w1a, w1b, b1)
        g = _sc_gather(p, q, dst, src)
        msg = _tc_edge(g, w2, b2, w3, b3, w4, b4)
        parts = _sc_scatter(msg, dst)
        x = _tc_update(x, parts[0], parts[1], w1a, w1b, b1,
                       w2, b2, w3, b3, w4, b4)
    return _tc_mlp4(x, params["node_out"])


# speedup vs baseline: 4.4671x; 1.4338x over previous
import functools

import jax
import jax.numpy as jnp
from jax import lax
from jax.experimental import pallas as pl
from jax.experimental.pallas import tpu as pltpu
from jax.experimental.pallas import tpu_sc as plsc

N = 10000
E = 320000
F = 128
NC, NS = 2, 16
NW = NC * NS
C = 128
NCHUNK = E // C
ROWS_PER_TILE = 624
TAIL_ROWS = N - NS * ROWS_PER_TILE
_PIECES = ((0, 128), (128, 128), (256, 128), (384, 128), (512, 112))

BR = 2000
BE = 1280


def _mm(h, w):
    return lax.dot_general(h, w, (((1,), (0,)), ((), ())),
                           preferred_element_type=jnp.float32)


def _full(shape):
    return pl.BlockSpec(shape, lambda i: (0,) * len(shape))


def _mlp4_body(x_ref, w1, b1, w2, b2, w3, b3, w4, b4, o_ref):
    h = x_ref[...]
    h = jnp.maximum(_mm(h, w1[...]) + b1[...], 0.0)
    h = jnp.maximum(_mm(h, w2[...]) + b2[...], 0.0)
    h = jnp.maximum(_mm(h, w3[...]) + b3[...], 0.0)
    o_ref[...] = jnp.maximum(_mm(h, w4[...]) + b4[...], 0.0)


def _tc_mlp4(x, ps):
    args = [x]
    in_specs = [pl.BlockSpec((BR, F), lambda i: (i, 0))]
    for w, b in ps:
        args += [w, b.reshape(1, F)]
        in_specs += [_full((F, F)), _full((1, F))]
    return pl.pallas_call(
        _mlp4_body,
        grid=(N // BR,),
        in_specs=in_specs,
        out_specs=pl.BlockSpec((BR, F), lambda i: (i, 0)),
        out_shape=jax.ShapeDtypeStruct((N, F), jnp.float32),
    )(*args)


def _pq_body(x_ref, wa, wb, b1, p_ref, q_ref):
    h = x_ref[...]
    p_ref[...] = _mm(h, wa[...]) + b1[...]
    q_ref[...] = _mm(h, wb[...])


def _tc_pq(x, wa, wb, b1):
    return pl.pallas_call(
        _pq_body,
        grid=(N // BR,),
        in_specs=[pl.BlockSpec((BR, F), lambda i: (i, 0)),
                  _full((F, F)), _full((F, F)), _full((1, F))],
        out_specs=(pl.BlockSpec((BR, F), lambda i: (i, 0)),
                   pl.BlockSpec((BR, F), lambda i: (i, 0))),
        out_shape=(jax.ShapeDtypeStruct((N, F), jnp.float32),
                   jax.ShapeDtypeStruct((N, F), jnp.float32)),
    )(x, wa, wb, b1.reshape(1, F))


def _edge_body(g_ref, w2, b2, w3, b3, w4, b4, o_ref):
    h = jnp.maximum(g_ref[...], 0.0)
    h = jnp.maximum(_mm(h, w2[...]) + b2[...], 0.0)
    h = jnp.maximum(_mm(h, w3[...]) + b3[...], 0.0)
    o_ref[...] = jnp.maximum(_mm(h, w4[...]) + b4[...], 0.0)


def _tc_edge(g, w2, b2, w3, b3, w4, b4):
    return pl.pallas_call(
        _edge_body,
        grid=(E // BE,),
        in_specs=[pl.BlockSpec((BE, F), lambda i: (i, 0)),
                  _full((F, F)), _full((1, F)),
                  _full((F, F)), _full((1, F)),
                  _full((F, F)), _full((1, F))],
        out_specs=pl.BlockSpec((BE, F), lambda i: (i, 0)),
        out_shape=jax.ShapeDtypeStruct((E, F), jnp.float32),
    )(g, w2, b2.reshape(1, F), w3, b3.reshape(1, F), w4, b4.reshape(1, F))


def _upd_body(x_ref, p0_ref, p1_ref, va, vb, b1, w2, b2, w3, b3, w4, b4, o_ref):
    x = x_ref[...]
    a = p0_ref[...] + p1_ref[...]
    h = jnp.maximum(_mm(x, va[...]) + _mm(a, vb[...]) + b1[...], 0.0)
    h = jnp.maximum(_mm(h, w2[...]) + b2[...], 0.0)
    h = jnp.maximum(_mm(h, w3[...]) + b3[...], 0.0)
    o_ref[...] = x + jnp.maximum(_mm(h, w4[...]) + b4[...], 0.0)


def _tc_update(x, p0, p1, va, vb, b1, w2, b2, w3, b3, w4, b4):
    row = pl.BlockSpec((BR, F), lambda i: (i, 0))
    return pl.pallas_call(
        _upd_body,
        grid=(N // BR,),
        in_specs=[row, row, row,
                  _full((F, F)), _full((F, F)), _full((1, F)),
                  _full((F, F)), _full((1, F)),
                  _full((F, F)), _full((1, F)),
                  _full((F, F)), _full((1, F))],
        out_specs=row,
        out_shape=jax.ShapeDtypeStruct((N, F), jnp.float32),
    )(x, p0, p1, va, vb, b1.reshape(1, F), w2, b2.reshape(1, F),
      w3, b3.reshape(1, F), w4, b4.reshape(1, F))


def _sc_mesh():
    return plsc.VectorSubcoreMesh(core_axis_name="c", subcore_axis_name="s",
                                  num_cores=NC, num_subcores=NS)


NFULL = NCHUNK // NW
NTAIL = NCHUNK - NFULL * NW


def _sc_gather(p, q, dst, src):
    @functools.partial(
        pl.kernel,
        out_type=jax.ShapeDtypeStruct((E, F), jnp.float32),
        mesh=_sc_mesh(),
        scratch_types=[
            [pltpu.VMEM((C,), jnp.int32)] * 2,
            [pltpu.VMEM((C,), jnp.int32)] * 2,
            [pltpu.VMEM((C, F), jnp.float32)] * 2,
            [pltpu.VMEM((C, F), jnp.float32)] * 2,
            [pltpu.SemaphoreType.DMA] * 2,
            [pltpu.SemaphoreType.DMA] * 2,
            [pltpu.SemaphoreType.DMA] * 2,
        ],
    )
    def k(p_hbm, q_hbm, dst_hbm, src_hbm, g_hbm, idx_d, idx_s, bufp, bufq,
          semi, semr, semw):
        wid = lax.axis_index("s") * NC + lax.axis_index("c")
        n = NFULL

        def cbase(i):
            return pl.multiple_of((wid + i * NW) * C, C)

        def idx_start(i, b):
            base = cbase(i)
            pltpu.async_copy(dst_hbm.at[pl.ds(base, C)], idx_d[b], semi[b])
            pltpu.async_copy(src_hbm.at[pl.ds(base, C)], idx_s[b], semi[b])

        def idx_wait(b):
            pltpu.make_async_copy(dst_hbm.at[pl.ds(0, C)], idx_d[b], semi[b]).wait()
            pltpu.make_async_copy(src_hbm.at[pl.ds(0, C)], idx_s[b], semi[b]).wait()

        def gather_start(b):
            pltpu.async_copy(p_hbm.at[idx_d[b]], bufp[b], semr[b])
            pltpu.async_copy(q_hbm.at[idx_s[b]], bufq[b], semr[b])

        def gather_wait(b):
            pltpu.make_async_copy(p_hbm.at[idx_d[b]], bufp[b], semr[b]).wait()
            pltpu.make_async_copy(q_hbm.at[idx_s[b]], bufq[b], semr[b]).wait()

        def wb_start(i, b):
            pltpu.async_copy(bufp[b], g_hbm.at[pl.ds(cbase(i), C)], semw[b])

        def wb_wait(b):
            pltpu.make_async_copy(bufp[b], g_hbm.at[pl.ds(0, C)], semw[b]).wait()

        def add(b):
            def row(r, c2):
                for kk in range(F // 16):
                    sl = pl.ds(kk * 16, 16)
                    bufp[b][r, sl] = bufp[b][r, sl] + bufq[b][r, sl]
                return c2

            lax.fori_loop(0, C, row, 0)

        idx_start(0, 0)
        idx_wait(0)
        gather_start(0)
        idx_start(1, 1)

        def step(i, b):
            nb = 1 - b
            gather_wait(b)

            @pl.when(i + 1 < n)
            def _():
                idx_wait(nb)

                @pl.when(i >= 1)
                def _():
                    wb_wait(nb)

                gather_start(nb)

            @pl.when(i + 2 < n)
            def _():
                idx_start(i + 2, b)

            add(b)
            wb_start(i, b)

        def group(g, carry):
            step(2 * g, 0)
            step(2 * g + 1, 1)
            return carry

        lax.fori_loop(0, n // 2, group, 0)
        wb_wait(0)
        wb_wait(1)

        @pl.when(wid < NTAIL)
        def _tail():
            base = pl.multiple_of((NFULL * NW + wid) * C, C)
            pltpu.sync_copy(dst_hbm.at[pl.ds(base, C)], idx_d[0])
            pltpu.sync_copy(src_hbm.at[pl.ds(base, C)], idx_s[0])
            gather_start(0)
            gather_wait(0)
            add(0)
            pltpu.sync_copy(bufp[0], g_hbm.at[pl.ds(base, C)])

    return k(p, q, dst, src)


def _sc_scatter(msg, dst):
    @functools.partial(
        pl.kernel,
        out_type=jax.ShapeDtypeStruct((NC, N, F), jnp.float32),
        mesh=_sc_mesh(),
        scratch_types=[
            [pltpu.VMEM((C,), jnp.int32)] * 2,
            [pltpu.VMEM((C, F), jnp.float32)] * 2,
            pltpu.VMEM_SHARED((N, F), jnp.float32),
            [pltpu.SemaphoreType.DMA] * 2,
            [pltpu.SemaphoreType.DMA] * 2,
        ],
    )
    def k(msg_hbm, dst_hbm, out_hbm, idx_v, buf, acc, semc, sems):
        cid = lax.axis_index("c")
        sid = lax.axis_index("s")
        wid = sid * NC + cid
        n = NFULL

        def zrow(r, c2):
            for kk in range(F // 16):
                buf[0][r, pl.ds(kk * 16, 16)] = jnp.zeros((16,), jnp.float32)
            return c2

        lax.fori_loop(0, C, zrow, 0)
        tile0 = sid * ROWS_PER_TILE
        for off, sz in _PIECES:
            r0 = pl.multiple_of(tile0 + off, 8)
            pltpu.sync_copy(buf[0].at[pl.ds(0, sz)], acc.at[pl.ds(r0, sz)])

        @pl.when(sid == 0)
        def _zero_tail():
            pltpu.sync_copy(buf[0].at[pl.ds(0, TAIL_ROWS)],
                            acc.at[pl.ds(NS * ROWS_PER_TILE, TAIL_ROWS)])

        plsc.subcore_barrier()

        def cbase(i):
            return pl.multiple_of((wid + i * NW) * C, C)

        def copy_start(i, b):
            base = cbase(i)
            pltpu.async_copy(dst_hbm.at[pl.ds(base, C)], idx_v[b], semc[b])
            pltpu.async_copy(msg_hbm.at[pl.ds(base, C)], buf[b], semc[b])

        def copy_wait(b):
            pltpu.make_async_copy(dst_hbm.at[pl.ds(0, C)], idx_v[b], semc[b]).wait()
            pltpu.make_async_copy(msg_hbm.at[pl.ds(0, C)], buf[b], semc[b]).wait()

        def scat_start(b):
            pltpu.async_copy(buf[b], acc.at[idx_v[b]], sems[b], add=True)

        def scat_wait(b):
            pltpu.make_async_copy(buf[b], acc.at[idx_v[b]], sems[b]).wait()

        copy_start(0, 0)

        def step(i, b):
            nb = 1 - b
            copy_wait(b)
            scat_start(b)

            @pl.when(i + 1 < n)
            def _():
                @pl.when(i >= 1)
                def _():
                    scat_wait(nb)

                copy_start(i + 1, nb)

        def group(g, carry):
            step(2 * g, 0)
            step(2 * g + 1, 1)
            return carry

        lax.fori_loop(0, n // 2, group, 0)
        scat_wait(0)
        scat_wait(1)

        @pl.when(wid < NTAIL)
        def _tail():
            base = pl.multiple_of((NFULL * NW + wid) * C, C)
            pltpu.sync_copy(dst_hbm.at[pl.ds(base, C)], idx_v[0])
            pltpu.sync_copy(msg_hbm.at[pl.ds(base, C)], buf[0])
            pltpu.sync_copy(buf[0], acc.at[idx_v[0]], add=True)

        plsc.subcore_barrier()

        for off, sz in _PIECES:
            r0 = pl.multiple_of(tile0 + off, 8)
            pltpu.sync_copy(acc.at[pl.ds(r0, sz)], out_hbm.at[cid, pl.ds(r0, sz)])

        @pl.when(sid == 0)
        def _write_tail():
            rt = NS * ROWS_PER_TILE
            pltpu.sync_copy(acc.at[pl.ds(rt, TAIL_ROWS)],
                            out_hbm.at[cid, pl.ds(rt, TAIL_ROWS)])

    return k(msg, dst)


def kernel(node_feature, params, edge_index):
    src = edge_index[0]
    dst = edge_index[1]
    x = _tc_mlp4(node_feature, params["node_in"])
    for lp in params["layers"]:
        (w1, b1), (w2, b2), (w3, b3), (w4, b4) = lp
        w1a, w1b = w1[:F], w1[F:]
        p, q = _tc_pq(x, w1a, w1b, b1)
        g = _sc_gather(p, q, dst, src)
        msg = _tc_edge(g, w2, b2, w3, b3, w4, b4)
        parts = _sc_scatter(msg, dst)
        x = _tc_update(x, parts[0], parts[1], w1a, w1b, b1,
                       w2, b2, w3, b3, w4, b4)
    return _tc_mlp4(x, params["node_out"])


# speedup vs baseline: 4.5927x; 1.0281x over previous
import functools

import jax
import jax.numpy as jnp
from jax import lax
from jax.experimental import pallas as pl
from jax.experimental.pallas import tpu as pltpu
from jax.experimental.pallas import tpu_sc as plsc

N = 10000
E = 320000
F = 128
NC, NS = 2, 16
NW = NC * NS
C = 128
NCHUNK = E // C
ROWS_PER_TILE = 624
TAIL_ROWS = N - NS * ROWS_PER_TILE
_PIECES = ((0, 128), (128, 128), (256, 128), (384, 128), (512, 112))

BR = 2000
BE = 1280


def _mm(h, w):
    return lax.dot_general(h, w, (((1,), (0,)), ((), ())),
                           preferred_element_type=jnp.float32)


def _full(shape):
    return pl.BlockSpec(shape, lambda i: (0,) * len(shape))


def _bmm(h, w):
    return lax.dot_general(h.astype(jnp.bfloat16), w.astype(jnp.bfloat16),
                           (((1,), (0,)), ((), ())),
                           preferred_element_type=jnp.float32)


def _apply_mlp4(h, ws, bs, pre=None):
    h = _mm(h, ws[0][...]) + bs[0][...]
    if pre is not None:
        h = h + pre
    h = jnp.maximum(h, 0.0)
    for w, b in zip(ws[1:], bs[1:]):
        h = jnp.maximum(_mm(h, w[...]) + b[...], 0.0)
    return h


def _mlp_args(ps):
    args, specs = [], []
    for w, b in ps:
        args += [w, b.reshape(1, F)]
        specs += [_full(w.shape), _full((1, F))]
    return args, specs


def _pq_from(h, w1_ref, b1_ref):
    p = _mm(h, w1_ref[0:F, :]) + b1_ref[...]
    q = _mm(h, w1_ref[F:2 * F, :])
    return p, q


def _in_pq_body(x_ref, w1, b1, w2, b2, w3, b3, w4, b4, lw1, lb1,
                x_out, p_out, q_out):
    ws, bs = (w1, w2, w3, w4), (b1, b2, b3, b4)
    h = _apply_mlp4(x_ref[...], ws, bs)
    x_out[...] = h
    p, q = _pq_from(h, lw1, lb1)
    p_out[...] = p
    q_out[...] = q


def _tc_in_pq(x, ps, lw1, lb1):
    args, specs = _mlp_args(ps)
    row = pl.BlockSpec((BR, F), lambda i: (i, 0))
    shp = jax.ShapeDtypeStruct((N, F), jnp.float32)
    return pl.pallas_call(
        _in_pq_body,
        grid=(N // BR,),
        in_specs=[row] + specs + [_full((2 * F, F)), _full((1, F))],
        out_specs=(row, row, row),
        out_shape=(shp, shp, shp),
    )(x, *args, lw1, lb1.reshape(1, F))


def _edge_body(g_ref, w2, b2, w3, b3, w4, b4, o_ref):
    h = jnp.maximum(g_ref[...], 0.0)
    h = jnp.maximum(_bmm(h, w2[...]) + b2[...], 0.0)
    h = jnp.maximum(_bmm(h, w3[...]) + b3[...], 0.0)
    o_ref[...] = jnp.maximum(_bmm(h, w4[...]) + b4[...], 0.0)


def _tc_edge(g, w2, b2, w3, b3, w4, b4):
    return pl.pallas_call(
        _edge_body,
        grid=(E // BE,),
        in_specs=[pl.BlockSpec((BE, F), lambda i: (i, 0)),
                  _full((F, F)), _full((1, F)),
                  _full((F, F)), _full((1, F)),
                  _full((F, F)), _full((1, F))],
        out_specs=pl.BlockSpec((BE, F), lambda i: (i, 0)),
        out_shape=jax.ShapeDtypeStruct((E, F), jnp.float32),
    )(g, w2, b2.reshape(1, F), w3, b3.reshape(1, F), w4, b4.reshape(1, F))


def _update_from(x, parts_ref, w1_ref, bs, ws_tail):
    a = parts_ref[0] + parts_ref[1]
    h = jnp.maximum(_mm(x, w1_ref[0:F, :]) + _mm(a, w1_ref[F:2 * F, :])
                    + bs[0][...], 0.0)
    for w, b in zip(ws_tail, bs[1:]):
        h = jnp.maximum(_mm(h, w[...]) + b[...], 0.0)
    return x + h


def _upd_pq_body(x_ref, parts_ref, w1, b1, w2, b2, w3, b3, w4, b4, nw1, nb1,
                 x_out, p_out, q_out):
    xn = _update_from(x_ref[...], parts_ref, w1, (b1, b2, b3, b4), (w2, w3, w4))
    x_out[...] = xn
    p, q = _pq_from(xn, nw1, nb1)
    p_out[...] = p
    q_out[...] = q


def _tc_upd_pq(x, parts, lp, nw1, nb1):
    (w1, b1), (w2, b2), (w3, b3), (w4, b4) = lp
    row = pl.BlockSpec((BR, F), lambda i: (i, 0))
    pspec = pl.BlockSpec((NC, BR, F), lambda i: (0, i, 0))
    shp = jax.ShapeDtypeStruct((N, F), jnp.float32)
    return pl.pallas_call(
        _upd_pq_body,
        grid=(N // BR,),
        in_specs=[row, pspec, _full((2 * F, F)), _full((1, F)),
                  _full((F, F)), _full((1, F)),
                  _full((F, F)), _full((1, F)),
                  _full((F, F)), _full((1, F)),
                  _full((2 * F, F)), _full((1, F))],
        out_specs=(row, row, row),
        out_shape=(shp, shp, shp),
    )(x, parts, w1, b1.reshape(1, F), w2, b2.reshape(1, F),
      w3, b3.reshape(1, F), w4, b4.reshape(1, F), nw1, nb1.reshape(1, F))


def _upd_out_body(x_ref, parts_ref, w1, b1, w2, b2, w3, b3, w4, b4,
                  o1, ob1, o2, ob2, o3, ob3, o4, ob4, out_ref):
    xn = _update_from(x_ref[...], parts_ref, w1, (b1, b2, b3, b4), (w2, w3, w4))
    out_ref[...] = _apply_mlp4(xn, (o1, o2, o3, o4), (ob1, ob2, ob3, ob4))


def _tc_upd_out(x, parts, lp, out_ps):
    (w1, b1), (w2, b2), (w3, b3), (w4, b4) = lp
    oargs, ospecs = _mlp_args(out_ps)
    row = pl.BlockSpec((BR, F), lambda i: (i, 0))
    pspec = pl.BlockSpec((NC, BR, F), lambda i: (0, i, 0))
    return pl.pallas_call(
        _upd_out_body,
        grid=(N // BR,),
        in_specs=[row, pspec, _full((2 * F, F)), _full((1, F)),
                  _full((F, F)), _full((1, F)),
                  _full((F, F)), _full((1, F)),
                  _full((F, F)), _full((1, F))] + ospecs,
        out_specs=row,
        out_shape=jax.ShapeDtypeStruct((N, F), jnp.float32),
    )(x, parts, w1, b1.reshape(1, F), w2, b2.reshape(1, F),
      w3, b3.reshape(1, F), w4, b4.reshape(1, F), *oargs)


def _sc_mesh():
    return plsc.VectorSubcoreMesh(core_axis_name="c", subcore_axis_name="s",
                                  num_cores=NC, num_subcores=NS)


NFULL = NCHUNK // NW
NTAIL = NCHUNK - NFULL * NW


def _sc_gather(p, q, dst, src):
    @functools.partial(
        pl.kernel,
        out_type=jax.ShapeDtypeStruct((E, F), jnp.float32),
        mesh=_sc_mesh(),
        scratch_types=[
            [pltpu.VMEM((C,), jnp.int32)] * 2,
            [pltpu.VMEM((C,), jnp.int32)] * 2,
            [pltpu.VMEM((C, F), jnp.float32)] * 2,
            [pltpu.VMEM((C, F), jnp.float32)] * 2,
            [pltpu.SemaphoreType.DMA] * 2,
            [pltpu.SemaphoreType.DMA] * 2,
            [pltpu.SemaphoreType.DMA] * 2,
        ],
    )
    def k(p_hbm, q_hbm, dst_hbm, src_hbm, g_hbm, idx_d, idx_s, bufp, bufq,
          semi, semr, semw):
        wid = lax.axis_index("s") * NC + lax.axis_index("c")
        n = NFULL

        def cbase(i):
            return pl.multiple_of((wid + i * NW) * C, C)

        def idx_start(i, b):
            base = cbase(i)
            pltpu.async_copy(dst_hbm.at[pl.ds(base, C)], idx_d[b], semi[b])
            pltpu.async_copy(src_hbm.at[pl.ds(base, C)], idx_s[b], semi[b])

        def idx_wait(b):
            pltpu.make_async_copy(dst_hbm.at[pl.ds(0, C)], idx_d[b], semi[b]).wait()
            pltpu.make_async_copy(src_hbm.at[pl.ds(0, C)], idx_s[b], semi[b]).wait()

        def gather_start(b):
            pltpu.async_copy(p_hbm.at[idx_d[b]], bufp[b], semr[b])
            pltpu.async_copy(q_hbm.at[idx_s[b]], bufq[b], semr[b])

        def gather_wait(b):
            pltpu.make_async_copy(p_hbm.at[idx_d[b]], bufp[b], semr[b]).wait()
            pltpu.make_async_copy(q_hbm.at[idx_s[b]], bufq[b], semr[b]).wait()

        def wb_start(i, b):
            pltpu.async_copy(bufp[b], g_hbm.at[pl.ds(cbase(i), C)], semw[b])

        def wb_wait(b):
            pltpu.make_async_copy(bufp[b], g_hbm.at[pl.ds(0, C)], semw[b]).wait()

        def add(b):
            def row(r, c2):
                for kk in range(F // 16):
                    sl = pl.ds(kk * 16, 16)
                    bufp[b][r, sl] = bufp[b][r, sl] + bufq[b][r, sl]
                return c2

            lax.fori_loop(0, C, row, 0)

        idx_start(0, 0)
        idx_wait(0)
        gather_start(0)
        idx_start(1, 1)

        def step(i, b):
            nb = 1 - b
            gather_wait(b)

            @pl.when(i + 1 < n)
            def _():
                idx_wait(nb)

                @pl.when(i >= 1)
                def _():
                    wb_wait(nb)

                gather_start(nb)

            @pl.when(i + 2 < n)
            def _():
                idx_start(i + 2, b)

            add(b)
            wb_start(i, b)

        def group(g, carry):
            step(2 * g, 0)
            step(2 * g + 1, 1)
            return carry

        lax.fori_loop(0, n // 2, group, 0)
        wb_wait(0)
        wb_wait(1)

        @pl.when(wid < NTAIL)
        def _tail():
            base = pl.multiple_of((NFULL * NW + wid) * C, C)
            pltpu.sync_copy(dst_hbm.at[pl.ds(base, C)], idx_d[0])
            pltpu.sync_copy(src_hbm.at[pl.ds(base, C)], idx_s[0])
            gather_start(0)
            gather_wait(0)
            add(0)
            pltpu.sync_copy(bufp[0], g_hbm.at[pl.ds(base, C)])

    return k(p, q, dst, src)


def _sc_scatter(msg, dst):
    @functools.partial(
        pl.kernel,
        out_type=jax.ShapeDtypeStruct((NC, N, F), jnp.float32),
        mesh=_sc_mesh(),
        scratch_types=[
            [pltpu.VMEM((C,), jnp.int32)] * 2,
            [pltpu.VMEM((C, F), jnp.float32)] * 2,
            pltpu.VMEM_SHARED((N, F), jnp.float32),
            [pltpu.SemaphoreType.DMA] * 2,
            [pltpu.SemaphoreType.DMA] * 2,
        ],
    )
    def k(msg_hbm, dst_hbm, out_hbm, idx_v, buf, acc, semc, sems):
        cid = lax.axis_index("c")
        sid = lax.axis_index("s")
        wid = sid * NC + cid
        n = NFULL

        def zrow(r, c2):
            for kk in range(F // 16):
                buf[0][r, pl.ds(kk * 16, 16)] = jnp.zeros((16,), jnp.float32)
            return c2

        lax.fori_loop(0, C, zrow, 0)
        tile0 = sid * ROWS_PER_TILE
        for off, sz in _PIECES:
            r0 = pl.multiple_of(tile0 + off, 8)
            pltpu.sync_copy(buf[0].at[pl.ds(0, sz)], acc.at[pl.ds(r0, sz)])

        @pl.when(sid == 0)
        def _zero_tail():
            pltpu.sync_copy(buf[0].at[pl.ds(0, TAIL_ROWS)],
                            acc.at[pl.ds(NS * ROWS_PER_TILE, TAIL_ROWS)])

        plsc.subcore_barrier()

        def cbase(i):
            return pl.multiple_of((wid + i * NW) * C, C)

        def copy_start(i, b):
            base = cbase(i)
            pltpu.async_copy(dst_hbm.at[pl.ds(base, C)], idx_v[b], semc[b])
            pltpu.async_copy(msg_hbm.at[pl.ds(base, C)], buf[b], semc[b])

        def copy_wait(b):
            pltpu.make_async_copy(dst_hbm.at[pl.ds(0, C)], idx_v[b], semc[b]).wait()
            pltpu.make_async_copy(msg_hbm.at[pl.ds(0, C)], buf[b], semc[b]).wait()

        def scat_start(b):
            pltpu.async_copy(buf[b], acc.at[idx_v[b]], sems[b], add=True)

        def scat_wait(b):
            pltpu.make_async_copy(buf[b], acc.at[idx_v[b]], sems[b]).wait()

        copy_start(0, 0)

        def step(i, b):
            nb = 1 - b
            copy_wait(b)
            scat_start(b)

            @pl.when(i + 1 < n)
            def _():
                @pl.when(i >= 1)
                def _():
                    scat_wait(nb)

                copy_start(i + 1, nb)

        def group(g, carry):
            step(2 * g, 0)
            step(2 * g + 1, 1)
            return carry

        lax.fori_loop(0, n // 2, group, 0)
        scat_wait(0)
        scat_wait(1)

        @pl.when(wid < NTAIL)
        def _tail():
            base = pl.multiple_of((NFULL * NW + wid) * C, C)
            pltpu.sync_copy(dst_hbm.at[pl.ds(base, C)], idx_v[0])
            pltpu.sync_copy(msg_hbm.at[pl.ds(base, C)], buf[0])
            pltpu.sync_copy(buf[0], acc.at[idx_v[0]], add=True)

        plsc.subcore_barrier()

        for off, sz in _PIECES:
            r0 = pl.multiple_of(tile0 + off, 8)
            pltpu.sync_copy(acc.at[pl.ds(r0, sz)], out_hbm.at[cid, pl.ds(r0, sz)])

        @pl.when(sid == 0)
        def _write_tail():
            rt = NS * ROWS_PER_TILE
            pltpu.sync_copy(acc.at[pl.ds(rt, TAIL_ROWS)],
                            out_hbm.at[cid, pl.ds(rt, TAIL_ROWS)])

    return k(msg, dst)


def kernel(node_feature, params, edge_index):
    src = edge_index[0]
    dst = edge_index[1]
    l0, l1 = params["layers"]
    x, p, q = _tc_in_pq(node_feature, params["node_in"], l0[0][0], l0[0][1])

    g = _sc_gather(p, q, dst, src)
    msg = _tc_edge(g, l0[1][0], l0[1][1], l0[2][0], l0[2][1],
                   l0[3][0], l0[3][1])
    parts = _sc_scatter(msg, dst)
    x, p, q = _tc_upd_pq(x, parts, l0, l1[0][0], l1[0][1])

    g = _sc_gather(p, q, dst, src)
    msg = _tc_edge(g, l1[1][0], l1[1][1], l1[2][0], l1[2][1],
                   l1[3][0], l1[3][1])
    parts = _sc_scatter(msg, dst)
    return _tc_upd_out(x, parts, l1, params["node_out"])


# speedup vs baseline: 5.4941x; 1.1963x over previous
import functools

import numpy as np

import jax
import jax.numpy as jnp
from jax import lax
from jax.experimental import pallas as pl
from jax.experimental.pallas import tpu as pltpu
from jax.experimental.pallas import tpu_sc as plsc

N = 10000
E = 320000
F = 128
NC, NS = 2, 16
NW = NC * NS
C = 128
NCHUNK = E // C
ROWS_PER_TILE = 624
TAIL_ROWS = N - NS * ROWS_PER_TILE
_PIECES = ((0, 128), (128, 128), (256, 128), (384, 128), (512, 112))

BR = 2000
BE = 1280

EH = E // 2


def _mm(h, w):
    return lax.dot_general(h, w, (((1,), (0,)), ((), ())),
                           preferred_element_type=jnp.float32)


def _full(shape):
    return pl.BlockSpec(shape, lambda i: (0,) * len(shape))


def _bmm(h, w):
    return lax.dot_general(h.astype(jnp.bfloat16), w.astype(jnp.bfloat16),
                           (((1,), (0,)), ((), ())),
                           preferred_element_type=jnp.float32)


def _apply_mlp4(h, ws, bs, pre=None):
    h = _mm(h, ws[0][...]) + bs[0][...]
    if pre is not None:
        h = h + pre
    h = jnp.maximum(h, 0.0)
    for w, b in zip(ws[1:], bs[1:]):
        h = jnp.maximum(_mm(h, w[...]) + b[...], 0.0)
    return h


def _mlp_args(ps):
    args, specs = [], []
    for w, b in ps:
        args += [w, b.reshape(1, F)]
        specs += [_full(w.shape), _full((1, F))]
    return args, specs


def _pq_from(h, w1_ref, b1_ref):
    p = _mm(h, w1_ref[0:F, :]) + b1_ref[...]
    q = _mm(h, w1_ref[F:2 * F, :])
    return p, q


def _in_pq_body(x_ref, w1, b1, w2, b2, w3, b3, w4, b4, lw1, lb1,
                x_out, p_out, q_out):
    ws, bs = (w1, w2, w3, w4), (b1, b2, b3, b4)
    h = _apply_mlp4(x_ref[...], ws, bs)
    x_out[...] = h
    p, q = _pq_from(h, lw1, lb1)
    p_out[...] = p
    q_out[...] = q


def _tc_in_pq(x, ps, lw1, lb1):
    args, specs = _mlp_args(ps)
    row = pl.BlockSpec((BR, F), lambda i: (i, 0))
    shp = jax.ShapeDtypeStruct((N, F), jnp.float32)
    return pl.pallas_call(
        _in_pq_body,
        grid=(N // BR,),
        in_specs=[row] + specs + [_full((2 * F, F)), _full((1, F))],
        out_specs=(row, row, row),
        out_shape=(shp, shp, shp),
    )(x, *args, lw1, lb1.reshape(1, F))


def _edge_body(g_ref, w2, b2, w3, b3, w4, b4, o_ref):
    h = jnp.maximum(g_ref[...], 0.0)
    h = jnp.maximum(_bmm(h, w2[...]) + b2[...], 0.0)
    h = jnp.maximum(_bmm(h, w3[...]) + b3[...], 0.0)
    o_ref[...] = jnp.maximum(_bmm(h, w4[...]) + b4[...], 0.0)


def _tc_edge(g, w2, b2, w3, b3, w4, b4):
    return pl.pallas_call(
        _edge_body,
        grid=(g.shape[0] // BE,),
        in_specs=[pl.BlockSpec((BE, F), lambda i: (i, 0)),
                  _full((F, F)), _full((1, F)),
                  _full((F, F)), _full((1, F)),
                  _full((F, F)), _full((1, F))],
        out_specs=pl.BlockSpec((BE, F), lambda i: (i, 0)),
        out_shape=jax.ShapeDtypeStruct((g.shape[0], F), jnp.float32),
    )(g, w2, b2.reshape(1, F), w3, b3.reshape(1, F), w4, b4.reshape(1, F))


def _update_from(x, pa_ref, pb_ref, w1_ref, bs, ws_tail):
    a = (pa_ref[0] + pa_ref[1]) + (pb_ref[0] + pb_ref[1])
    h = jnp.maximum(_mm(x, w1_ref[0:F, :]) + _mm(a, w1_ref[F:2 * F, :])
                    + bs[0][...], 0.0)
    for w, b in zip(ws_tail, bs[1:]):
        h = jnp.maximum(_mm(h, w[...]) + b[...], 0.0)
    return x + h


def _upd_pq_body(x_ref, pa_ref, pb_ref, w1, b1, w2, b2, w3, b3, w4, b4,
                 nw1, nb1, x_out, p_out, q_out):
    xn = _update_from(x_ref[...], pa_ref, pb_ref, w1,
                      (b1, b2, b3, b4), (w2, w3, w4))
    x_out[...] = xn
    p, q = _pq_from(xn, nw1, nb1)
    p_out[...] = p
    q_out[...] = q


def _tc_upd_pq(x, parts_a, parts_b, lp, nw1, nb1):
    (w1, b1), (w2, b2), (w3, b3), (w4, b4) = lp
    row = pl.BlockSpec((BR, F), lambda i: (i, 0))
    pspec = pl.BlockSpec((NC, BR, F), lambda i: (0, i, 0))
    shp = jax.ShapeDtypeStruct((N, F), jnp.float32)
    return pl.pallas_call(
        _upd_pq_body,
        grid=(N // BR,),
        in_specs=[row, pspec, pspec, _full((2 * F, F)), _full((1, F)),
                  _full((F, F)), _full((1, F)),
                  _full((F, F)), _full((1, F)),
                  _full((F, F)), _full((1, F)),
                  _full((2 * F, F)), _full((1, F))],
        out_specs=(row, row, row),
        out_shape=(shp, shp, shp),
    )(x, parts_a, parts_b, w1, b1.reshape(1, F), w2, b2.reshape(1, F),
      w3, b3.reshape(1, F), w4, b4.reshape(1, F), nw1, nb1.reshape(1, F))


def _upd_out_body(x_ref, pa_ref, pb_ref, w1, b1, w2, b2, w3, b3, w4, b4,
                  o1, ob1, o2, ob2, o3, ob3, o4, ob4, out_ref):
    xn = _update_from(x_ref[...], pa_ref, pb_ref, w1,
                      (b1, b2, b3, b4), (w2, w3, w4))
    out_ref[...] = _apply_mlp4(xn, (o1, o2, o3, o4), (ob1, ob2, ob3, ob4))


def _tc_upd_out(x, parts_a, parts_b, lp, out_ps):
    (w1, b1), (w2, b2), (w3, b3), (w4, b4) = lp
    oargs, ospecs = _mlp_args(out_ps)
    row = pl.BlockSpec((BR, F), lambda i: (i, 0))
    pspec = pl.BlockSpec((NC, BR, F), lambda i: (0, i, 0))
    return pl.pallas_call(
        _upd_out_body,
        grid=(N // BR,),
        in_specs=[row, pspec, pspec, _full((2 * F, F)), _full((1, F)),
                  _full((F, F)), _full((1, F)),
                  _full((F, F)), _full((1, F)),
                  _full((F, F)), _full((1, F))] + ospecs,
        out_specs=row,
        out_shape=jax.ShapeDtypeStruct((N, F), jnp.float32),
    )(x, parts_a, parts_b, w1, b1.reshape(1, F), w2, b2.reshape(1, F),
      w3, b3.reshape(1, F), w4, b4.reshape(1, F), *oargs)


def _sc_mesh():
    return plsc.VectorSubcoreMesh(core_axis_name="c", subcore_axis_name="s",
                                  num_cores=NC, num_subcores=NS)


def _sc_gather(p, q, dst, src, co, ne):
    nchunk = ne // C
    nfull = nchunk // NW
    ntail = nchunk - nfull * NW

    @functools.partial(
        pl.kernel,
        out_type=jax.ShapeDtypeStruct((ne, F), jnp.float32),
        mesh=_sc_mesh(),
        scratch_types=[
            [pltpu.VMEM((C,), jnp.int32)] * 2,
            [pltpu.VMEM((C,), jnp.int32)] * 2,
            [pltpu.VMEM((C, F), jnp.float32)] * 2,
            [pltpu.VMEM((C, F), jnp.float32)] * 2,
            [pltpu.SemaphoreType.DMA] * 2,
            [pltpu.SemaphoreType.DMA] * 2,
            [pltpu.SemaphoreType.DMA] * 2,
        ],
    )
    def k(p_hbm, q_hbm, dst_hbm, src_hbm, g_hbm, idx_d, idx_s, bufp, bufq,
          semi, semr, semw):
        wid = lax.axis_index("s") * NC + lax.axis_index("c")
        n = nfull

        def cbase(i):
            return pl.multiple_of((co + wid + i * NW) * C, C)

        def obase(i):
            return pl.multiple_of((wid + i * NW) * C, C)

        def idx_start(i, b):
            base = cbase(i)
            pltpu.async_copy(dst_hbm.at[pl.ds(base, C)], idx_d[b], semi[b])
            pltpu.async_copy(src_hbm.at[pl.ds(base, C)], idx_s[b], semi[b])

        def idx_wait(b):
            pltpu.make_async_copy(dst_hbm.at[pl.ds(0, C)], idx_d[b], semi[b]).wait()
            pltpu.make_async_copy(src_hbm.at[pl.ds(0, C)], idx_s[b], semi[b]).wait()

        def gather_start(b):
            pltpu.async_copy(p_hbm.at[idx_d[b]], bufp[b], semr[b])
            pltpu.async_copy(q_hbm.at[idx_s[b]], bufq[b], semr[b])

        def gather_wait(b):
            pltpu.make_async_copy(p_hbm.at[idx_d[b]], bufp[b], semr[b]).wait()
            pltpu.make_async_copy(q_hbm.at[idx_s[b]], bufq[b], semr[b]).wait()

        def wb_start(i, b):
            pltpu.async_copy(bufp[b], g_hbm.at[pl.ds(obase(i), C)], semw[b])

        def wb_wait(b):
            pltpu.make_async_copy(bufp[b], g_hbm.at[pl.ds(0, C)], semw[b]).wait()

        def add(b):
            def row(r, c2):
                for kk in range(F // 16):
                    sl = pl.ds(kk * 16, 16)
                    bufp[b][r, sl] = bufp[b][r, sl] + bufq[b][r, sl]
                return c2

            lax.fori_loop(0, C, row, 0)

        idx_start(0, 0)
        idx_wait(0)
        gather_start(0)
        idx_start(1, 1)

        def step(i, b):
            nb = 1 - b
            gather_wait(b)

            @pl.when(i + 1 < n)
            def _():
                idx_wait(nb)

                @pl.when(i >= 1)
                def _():
                    wb_wait(nb)

                gather_start(nb)

            @pl.when(i + 2 < n)
            def _():
                idx_start(i + 2, b)

            add(b)
            wb_start(i, b)

        def group(g, carry):
            step(2 * g, 0)
            step(2 * g + 1, 1)
            return carry

        lax.fori_loop(0, n // 2, group, 0)
        if n % 2:
            step(n - 1, 0)
        wb_wait(0)
        wb_wait(1)

        @pl.when(wid < ntail)
        def _tail():
            base = pl.multiple_of((co + nfull * NW + wid) * C, C)
            ob = pl.multiple_of((nfull * NW + wid) * C, C)
            pltpu.sync_copy(dst_hbm.at[pl.ds(base, C)], idx_d[0])
            pltpu.sync_copy(src_hbm.at[pl.ds(base, C)], idx_s[0])
            gather_start(0)
            gather_wait(0)
            add(0)
            pltpu.sync_copy(bufp[0], g_hbm.at[pl.ds(ob, C)])

    return k(p, q, dst, src)


def _sc_scatter(msg, dst, co, ne):
    nchunk = ne // C
    nfull = nchunk // NW
    ntail = nchunk - nfull * NW

    @functools.partial(
        pl.kernel,
        out_type=jax.ShapeDtypeStruct((NC, N, F), jnp.float32),
        mesh=_sc_mesh(),
        scratch_types=[
            [pltpu.VMEM((C,), jnp.int32)] * 2,
            [pltpu.VMEM((C, F), jnp.float32)] * 2,
            pltpu.VMEM_SHARED((N, F), jnp.float32),
            [pltpu.SemaphoreType.DMA] * 2,
            [pltpu.SemaphoreType.DMA] * 2,
        ],
    )
    def k(msg_hbm, dst_hbm, out_hbm, idx_v, buf, acc, semc, sems):
        cid = lax.axis_index("c")
        sid = lax.axis_index("s")
        wid = sid * NC + cid
        n = nfull

        def zrow(r, c2):
            for kk in range(F // 16):
                buf[0][r, pl.ds(kk * 16, 16)] = jnp.zeros((16,), jnp.float32)
            return c2

        lax.fori_loop(0, C, zrow, 0)
        tile0 = sid * ROWS_PER_TILE
        for off, sz in _PIECES:
            r0 = pl.multiple_of(tile0 + off, 8)
            pltpu.sync_copy(buf[0].at[pl.ds(0, sz)], acc.at[pl.ds(r0, sz)])

        @pl.when(sid == 0)
        def _zero_tail():
            pltpu.sync_copy(buf[0].at[pl.ds(0, TAIL_ROWS)],
                            acc.at[pl.ds(NS * ROWS_PER_TILE, TAIL_ROWS)])

        plsc.subcore_barrier()

        def cbase(i):
            return pl.multiple_of((co + wid + i * NW) * C, C)

        def obase(i):
            return pl.multiple_of((wid + i * NW) * C, C)

        def copy_start(i, b):
            pltpu.async_copy(dst_hbm.at[pl.ds(cbase(i), C)], idx_v[b], semc[b])
            pltpu.async_copy(msg_hbm.at[pl.ds(obase(i), C)], buf[b], semc[b])

        def copy_wait(b):
            pltpu.make_async_copy(dst_hbm.at[pl.ds(0, C)], idx_v[b], semc[b]).wait()
            pltpu.make_async_copy(msg_hbm.at[pl.ds(0, C)], buf[b], semc[b]).wait()

        def scat_start(b):
            pltpu.async_copy(buf[b], acc.at[idx_v[b]], sems[b], add=True)

        def scat_wait(b):
            pltpu.make_async_copy(buf[b], acc.at[idx_v[b]], sems[b]).wait()

        copy_start(0, 0)

        def step(i, b):
            nb = 1 - b
            copy_wait(b)
            scat_start(b)

            @pl.when(i + 1 < n)
            def _():
                @pl.when(i >= 1)
                def _():
                    scat_wait(nb)

                copy_start(i + 1, nb)

        def group(g, carry):
            step(2 * g, 0)
            step(2 * g + 1, 1)
            return carry

        lax.fori_loop(0, n // 2, group, 0)
        if n % 2:
            step(n - 1, 0)
        scat_wait(0)
        scat_wait(1)

        @pl.when(wid < ntail)
        def _tail():
            base = pl.multiple_of((co + nfull * NW + wid) * C, C)
            ob = pl.multiple_of((nfull * NW + wid) * C, C)
            pltpu.sync_copy(dst_hbm.at[pl.ds(base, C)], idx_v[0])
            pltpu.sync_copy(msg_hbm.at[pl.ds(ob, C)], buf[0])
            pltpu.sync_copy(buf[0], acc.at[idx_v[0]], add=True)

        plsc.subcore_barrier()

        for off, sz in _PIECES:
            r0 = pl.multiple_of(tile0 + off, 8)
            pltpu.sync_copy(acc.at[pl.ds(r0, sz)], out_hbm.at[cid, pl.ds(r0, sz)])

        @pl.when(sid == 0)
        def _write_tail():
            rt = NS * ROWS_PER_TILE
            pltpu.sync_copy(acc.at[pl.ds(rt, TAIL_ROWS)],
                            out_hbm.at[cid, pl.ds(rt, TAIL_ROWS)])

    return k(msg, dst)


def _layer_edges(p, q, dst, src, lp):
    (_, _), (w2, b2), (w3, b3), (w4, b4) = lp
    coB = EH // C
    ga = _sc_gather(p, q, dst, src, 0, EH)
    gb = _sc_gather(p, q, dst, src, coB, EH)
    ma = _tc_edge(ga, w2, b2, w3, b3, w4, b4)
    mb = _tc_edge(gb, w2, b2, w3, b3, w4, b4)
    pa = _sc_scatter(ma, dst, 0, EH)
    pb = _sc_scatter(mb, dst, coB, EH)
    return pa, pb


def kernel(node_feature, params, edge_index):
    src = edge_index[0]
    dst = edge_index[1]
    l0, l1 = params["layers"]
    x, p, q = _tc_in_pq(node_feature, params["node_in"], l0[0][0], l0[0][1])

    pa, pb = _layer_edges(p, q, dst, src, l0)
    x, p, q = _tc_upd_pq(x, pa, pb, l0, l1[0][0], l1[0][1])

    pa, pb = _layer_edges(p, q, dst, src, l1)
    return _tc_upd_out(x, pa, pb, l1, params["node_out"])
